# baseline (speedup 1.0000x reference)
import functools

import jax
import jax.numpy as jnp
from jax import lax
from jax.experimental import pallas as pl
from jax.experimental.pallas import tpu as pltpu
from jax.experimental.pallas import tpu_sc as plsc

B = 16384
D = 128
NC = 2
NS = 16
NW = NC * NS
PER_W = B // NW
C = 128
NCHUNK = PER_W // C
L = 16

_mesh = plsc.VectorSubcoreMesh(core_axis_name="c", subcore_axis_name="s")


@functools.partial(
    pl.kernel,
    mesh=_mesh,
    out_type=jax.ShapeDtypeStruct((B,), jnp.float32),
    compiler_params=pltpu.CompilerParams(needs_layout_passes=False),
    scratch_types=[
        pltpu.VMEM((PER_W,), jnp.int32),
        pltpu.VMEM((PER_W,), jnp.int32),
        pltpu.VMEM((2, C, D), jnp.float32),
        pltpu.VMEM((2, C, D), jnp.float32),
        pltpu.VMEM((PER_W,), jnp.float32),
        pltpu.VMEM((L * L,), jnp.float32),
        pltpu.SemaphoreType.DMA,
        pltpu.SemaphoreType.DMA,
        pltpu.SemaphoreType.DMA,
        pltpu.SemaphoreType.DMA,
    ],
)
def _mf_sc(uid_hbm, iid_hbm, utab_hbm, itab_hbm, out_hbm,
           idx_u, idx_i, rows_u, rows_i, out_v, tbuf,
           sem_u0, sem_u1, sem_i0, sem_i1):
    wid = lax.axis_index("s") * NC + lax.axis_index("c")
    base = wid * PER_W
    colbase = lax.iota(jnp.int32, L) * L
    sems_u = (sem_u0, sem_u1)
    sems_i = (sem_i0, sem_i1)

    pltpu.sync_copy(uid_hbm.at[pl.ds(base, PER_W)], idx_u)
    pltpu.sync_copy(iid_hbm.at[pl.ds(base, PER_W)], idx_i)

    subs = [(0, 128), (128, 128), (256, 128), (384, 96), (480, 32)]

    def fire(k):
        off, sz = subs[k]
        b = k % 2
        return (
            pltpu.async_copy(utab_hbm.at[idx_u.at[pl.ds(off, sz)]],
                             rows_u.at[b, pl.ds(0, sz)], sems_u[b]),
            pltpu.async_copy(itab_hbm.at[idx_i.at[pl.ds(off, sz)]],
                             rows_i.at[b, pl.ds(0, sz)], sems_i[b]),
        )

    pending = fire(0)
    for k in range(len(subs)):
        off, sz = subs[k]
        b = k % 2
        du, di = pending
        du.wait()
        di.wait()
        if k + 1 < len(subs):
            pending = fire(k + 1)
        ru = rows_u.at[b]
        ri = rows_i.at[b]

        def _group(g, _, off=off, ru=ru, ri=ri):
            for l in range(L):
                r = g * L + l
                acc = ru[r, pl.ds(0, L)] * ri[r, pl.ds(0, L)]
                for j in range(1, D // L):
                    acc = acc + ru[r, pl.ds(j * L, L)] * ri[r, pl.ds(j * L, L)]
                tbuf[pl.ds(l * L, L)] = acc
            out_vec = plsc.load_gather(tbuf, [colbase])
            for l in range(1, L):
                out_vec = out_vec + plsc.load_gather(tbuf, [colbase + l])
            out_v[pl.ds(off + g * L, L)] = 1.0 / (1.0 + jnp.exp(-out_vec))
            return 0

        lax.fori_loop(0, sz // L, _group, 0)

    pltpu.sync_copy(out_v, out_hbm.at[pl.ds(base, PER_W)])


def kernel(user_ids, item_ids, user_table, item_table):
    return _mf_sc(user_ids, item_ids, user_table, item_table)
